# baseline (speedup 1.0000x reference)
import functools

import jax
import jax.numpy as jnp
from jax import lax
from jax.experimental import pallas as pl
from jax.experimental.pallas import tpu as pltpu
from jax.experimental.pallas import tpu_sc as plsc

_P = 3
_Q = 3
_OUT_U = 256
_OUT_V = 256
_DIM = 3
_M = 64
_NCOL = 256
_ROWS_PER_W = 128
_JCH = 16
_U = 4


def _sc_body(ctrl_hbm, ub256_hbm, nurep_hbm, vb4_hbm, nvt_hbm, out_hbm,
             ctrl_v, ub256_v, nurep_v, vb4_v, nvt_v,
             t0, t1, t2, t3, buf0, buf1, buf2, buf3,
             sem0, sem1, sem2, sem3):
    wid = lax.axis_index("s") * 2 + lax.axis_index("c")
    b = wid // 2
    i0 = (wid % 2) * _ROWS_PER_W
    ts = (t0, t1, t2, t3)
    bufs = (buf0, buf1, buf2, buf3)
    sems = (sem0, sem1, sem2, sem3)

    pltpu.sync_copy(ctrl_hbm.at[b], ctrl_v)
    pltpu.sync_copy(ub256_hbm.at[pl.ds(i0 * 16, _ROWS_PER_W * 16)], ub256_v)
    pltpu.sync_copy(nurep_hbm.at[pl.ds(i0 * 64, _ROWS_PER_W * 64)], nurep_v)
    pltpu.sync_copy(vb4_hbm, vb4_v)
    pltpu.sync_copy(nvt_hbm, nvt_v)

    iota = lax.iota(jnp.int32, 16)
    iota3 = iota * 3

    def _stage_a(i_local, t_v):
        ub256 = ub256_v[pl.ds(i_local * 16, 16)]
        base = ub256 + iota
        nus = [nurep_v[pl.ds(i_local * 64 + 16 * l, 16)]
               for l in range(_P + 1)]
        for k in range(_NCOL // 16):
            acc = nus[0] * plsc.load_gather(ctrl_v, [base + (16 * k)])
            for l in range(1, _P + 1):
                acc = acc + nus[l] * plsc.load_gather(
                    ctrl_v, [base + (l * 256 + 16 * k)])
            t_v[pl.ds(16 * k, 16)] = acc

    def _iter(p, carry):
        @pl.when(p >= 1)
        def _drain():
            for s in range(_U):
                pltpu.make_async_copy(
                    bufs[s], out_hbm.at[b, i0 + _U * p - _U + s, :],
                    sems[s]).wait()
        for s in range(_U):
            _stage_a(_U * p + s, ts[s])
        for kc in range(_JCH):
            vb4 = vb4_v[pl.ds(16 * kc, 16)]
            nvs = [nvt_v[pl.ds(256 * r + 16 * kc, 16)]
                   for r in range(_Q + 1)]
            jbase = iota3 + (48 * kc)
            for s in range(_U):
                t_v = ts[s]
                accs = []
                for c in range(4):
                    a = nvs[0] * plsc.load_gather(t_v, [vb4 + c])
                    for r in range(1, _Q + 1):
                        a = a + nvs[r] * plsc.load_gather(
                            t_v, [vb4 + (4 * r + c)])
                    accs.append(a)
                w = accs[3]
                for c in range(_DIM):
                    plsc.store_scatter(bufs[s], [jbase + c], accs[c] / w)
        for s in range(_U):
            pltpu.async_copy(bufs[s], out_hbm.at[b, i0 + _U * p + s, :],
                             sems[s])
        return carry

    lax.fori_loop(0, _ROWS_PER_W // _U, _iter, 0)
    for s in range(_U):
        pltpu.make_async_copy(
            bufs[s], out_hbm.at[b, i0 + _ROWS_PER_W - _U + s, :],
            sems[s]).wait()


def kernel(ctrl_pts, Nu_uv, Nv_uv, uspan_uv, vspan_uv):
    batch, m, n, dimp1 = ctrl_pts.shape
    ctrl2 = ctrl_pts.reshape(batch, m * n * dimp1)

    ub = (uspan_uv - _P).astype(jnp.int32)
    vb = (vspan_uv - _Q).astype(jnp.int32)
    ub256 = jnp.broadcast_to((ub * _NCOL)[:, None], (_OUT_U, 16)).reshape(-1)
    nurep = jnp.broadcast_to(Nu_uv.astype(jnp.float32)[:, :, None],
                             (_OUT_U, _P + 1, 16)).reshape(-1)
    vb4c = vb * 4
    nvtc = Nv_uv.astype(jnp.float32).T.reshape(-1)

    run = functools.partial(
        pl.kernel,
        mesh=plsc.VectorSubcoreMesh(core_axis_name="c", subcore_axis_name="s"),
        compiler_params=pltpu.CompilerParams(needs_layout_passes=False),
        out_type=jax.ShapeDtypeStruct((batch, _OUT_U, _DIM * _OUT_V),
                                      jnp.float32),
        scratch_types=[
            pltpu.VMEM((_M * _NCOL,), jnp.float32),
            pltpu.VMEM((_ROWS_PER_W * 16,), jnp.int32),
            pltpu.VMEM((_ROWS_PER_W * (_P + 1) * 16,), jnp.float32),
            pltpu.VMEM((_OUT_V,), jnp.int32),
            pltpu.VMEM(((_Q + 1) * _OUT_V,), jnp.float32),
            pltpu.VMEM((_NCOL,), jnp.float32),
            pltpu.VMEM((_NCOL,), jnp.float32),
            pltpu.VMEM((_NCOL,), jnp.float32),
            pltpu.VMEM((_NCOL,), jnp.float32),
            pltpu.VMEM((_DIM * _OUT_V,), jnp.float32),
            pltpu.VMEM((_DIM * _OUT_V,), jnp.float32),
            pltpu.VMEM((_DIM * _OUT_V,), jnp.float32),
            pltpu.VMEM((_DIM * _OUT_V,), jnp.float32),
            pltpu.SemaphoreType.DMA,
            pltpu.SemaphoreType.DMA,
            pltpu.SemaphoreType.DMA,
            pltpu.SemaphoreType.DMA,
        ],
    )(_sc_body)
    out3 = run(ctrl2, ub256, nurep, vb4c, nvtc)
    return out3.reshape(batch, _OUT_U, _OUT_V, dimp1 - 1)


# speedup vs baseline: 1.2871x; 1.2871x over previous
import functools

import jax
import jax.numpy as jnp
from jax import lax
from jax.experimental import pallas as pl
from jax.experimental.pallas import tpu as pltpu
from jax.experimental.pallas import tpu_sc as plsc

_P = 3
_Q = 3
_OUT_U = 256
_OUT_V = 256
_DIM = 3
_M = 64
_NCOL = 256
_ROWS_PER_W = 128
_JCH = 16
_TSTRIDE = 1025
_TSIZE = 3 * _TSTRIDE + _NCOL


def _sc_body(ctrl_hbm, ub256_hbm, nurep_hbm, vb4_hbm, nvt_hbm, out_hbm,
             ctrl_v, ub256_v, nurep_v, vb4_v, nvt_v, t_a, t_b, buf_a, buf_b,
             sem_a, sem_b):
    wid = lax.axis_index("s") * 2 + lax.axis_index("c")
    b = wid // 2
    i0 = (wid % 2) * _ROWS_PER_W

    pltpu.sync_copy(ctrl_hbm.at[b], ctrl_v)
    pltpu.sync_copy(ub256_hbm.at[pl.ds(i0 * 16, _ROWS_PER_W * 16)], ub256_v)
    pltpu.sync_copy(nurep_hbm.at[pl.ds(i0 * 64, _ROWS_PER_W * 64)], nurep_v)
    pltpu.sync_copy(vb4_hbm, vb4_v)
    pltpu.sync_copy(nvt_hbm, nvt_v)

    iota = lax.iota(jnp.int32, 16)
    iota3 = iota * 3
    qofs = (iota & 3) * _TSTRIDE

    def _stage_a(i_local, t_v):
        ub256 = ub256_v[pl.ds(i_local * 16, 16)]
        base = ub256 + iota
        nus = [nurep_v[pl.ds(i_local * 64 + 16 * l, 16)]
               for l in range(_P + 1)]
        for k in range(_NCOL // 16):
            acc = nus[0] * plsc.load_gather(ctrl_v, [base + (16 * k)])
            for l in range(1, _P + 1):
                acc = acc + nus[l] * plsc.load_gather(
                    ctrl_v, [base + (l * 256 + 16 * k)])
            for q in range(4):
                t_v[pl.ds(q * _TSTRIDE + 16 * k, 16)] = acc

    def _pair(p, carry):
        @pl.when(p >= 1)
        def _drain():
            pltpu.make_async_copy(
                buf_a, out_hbm.at[b, i0 + 2 * p - 2, :], sem_a).wait()
            pltpu.make_async_copy(
                buf_b, out_hbm.at[b, i0 + 2 * p - 1, :], sem_b).wait()
        _stage_a(2 * p, t_a)
        _stage_a(2 * p + 1, t_b)
        for kc in range(_JCH):
            vb4 = vb4_v[pl.ds(16 * kc, 16)]
            nvs = [nvt_v[pl.ds(256 * r + 16 * kc, 16)]
                   for r in range(_Q + 1)]
            jbase = iota3 + (48 * kc)
            vb4q = vb4 + qofs
            for t_v, buf in ((t_a, buf_a), (t_b, buf_b)):
                accs = []
                for c in range(4):
                    a = nvs[0] * plsc.load_gather(t_v, [vb4q + c])
                    for r in range(1, _Q + 1):
                        a = a + nvs[r] * plsc.load_gather(
                            t_v, [vb4q + (4 * r + c)])
                    accs.append(a)
                w = accs[3]
                for c in range(_DIM):
                    plsc.store_scatter(buf, [jbase + c], accs[c] / w)
        pltpu.async_copy(buf_a, out_hbm.at[b, i0 + 2 * p, :], sem_a)
        pltpu.async_copy(buf_b, out_hbm.at[b, i0 + 2 * p + 1, :], sem_b)
        return carry

    lax.fori_loop(0, _ROWS_PER_W // 2, _pair, 0)
    pltpu.make_async_copy(
        buf_a, out_hbm.at[b, i0 + _ROWS_PER_W - 2, :], sem_a).wait()
    pltpu.make_async_copy(
        buf_b, out_hbm.at[b, i0 + _ROWS_PER_W - 1, :], sem_b).wait()


def kernel(ctrl_pts, Nu_uv, Nv_uv, uspan_uv, vspan_uv):
    batch, m, n, dimp1 = ctrl_pts.shape
    ctrl2 = ctrl_pts.reshape(batch, m * n * dimp1)

    ub = (uspan_uv - _P).astype(jnp.int32)
    vb = (vspan_uv - _Q).astype(jnp.int32)
    ub256 = jnp.broadcast_to((ub * _NCOL)[:, None], (_OUT_U, 16)).reshape(-1)
    nurep = jnp.broadcast_to(Nu_uv.astype(jnp.float32)[:, :, None],
                             (_OUT_U, _P + 1, 16)).reshape(-1)
    vb4c = vb * 4
    nvtc = Nv_uv.astype(jnp.float32).T.reshape(-1)

    run = functools.partial(
        pl.kernel,
        mesh=plsc.VectorSubcoreMesh(core_axis_name="c", subcore_axis_name="s"),
        compiler_params=pltpu.CompilerParams(needs_layout_passes=False),
        out_type=jax.ShapeDtypeStruct((batch, _OUT_U, _DIM * _OUT_V),
                                      jnp.float32),
        scratch_types=[
            pltpu.VMEM((_M * _NCOL,), jnp.float32),
            pltpu.VMEM((_ROWS_PER_W * 16,), jnp.int32),
            pltpu.VMEM((_ROWS_PER_W * (_P + 1) * 16,), jnp.float32),
            pltpu.VMEM((_OUT_V,), jnp.int32),
            pltpu.VMEM(((_Q + 1) * _OUT_V,), jnp.float32),
            pltpu.VMEM((_TSIZE,), jnp.float32),
            pltpu.VMEM((_TSIZE,), jnp.float32),
            pltpu.VMEM((_DIM * _OUT_V,), jnp.float32),
            pltpu.VMEM((_DIM * _OUT_V,), jnp.float32),
            pltpu.SemaphoreType.DMA,
            pltpu.SemaphoreType.DMA,
        ],
    )(_sc_body)
    out3 = run(ctrl2, ub256, nurep, vb4c, nvtc)
    return out3.reshape(batch, _OUT_U, _OUT_V, dimp1 - 1)


# speedup vs baseline: 1.4607x; 1.1349x over previous
import functools

import jax
import jax.numpy as jnp
from jax import lax
from jax.experimental import pallas as pl
from jax.experimental.pallas import tpu as pltpu
from jax.experimental.pallas import tpu_sc as plsc

_P = 3
_Q = 3
_OUT_U = 256
_OUT_V = 256
_DIM = 3
_M = 64
_NCOL = 256
_ROWS_PER_W = 128
_JCH = 16


def _sc_body(ctrl_hbm, ub256_hbm, nurep_hbm, vb4_hbm, nvt_hbm, out_hbm,
             ctrl_v, ub256_v, nurep_v, vb4_v, nvt_v, t_a, t_b, buf_a, buf_b,
             sem_a, sem_b):
    wid = lax.axis_index("s") * 2 + lax.axis_index("c")
    b = wid // 2
    i0 = (wid % 2) * _ROWS_PER_W

    pltpu.sync_copy(ctrl_hbm.at[b], ctrl_v)
    pltpu.sync_copy(ub256_hbm.at[pl.ds(i0 * 16, _ROWS_PER_W * 16)], ub256_v)
    pltpu.sync_copy(nurep_hbm.at[pl.ds(i0 * 64, _ROWS_PER_W * 64)], nurep_v)
    pltpu.sync_copy(vb4_hbm, vb4_v)
    pltpu.sync_copy(nvt_hbm, nvt_v)

    iota = lax.iota(jnp.int32, 16)
    iota3 = iota * 3

    def _stage_a(i_local, t_v):
        ub256 = ub256_v[pl.ds(i_local * 16, 16)]
        base = ub256 + iota
        nus = [nurep_v[pl.ds(i_local * 64 + 16 * l, 16)]
               for l in range(_P + 1)]
        for k in range(_NCOL // 16):
            acc = nus[0] * plsc.load_gather(ctrl_v, [base + (16 * k)])
            for l in range(1, _P + 1):
                acc = acc + nus[l] * plsc.load_gather(
                    ctrl_v, [base + (l * 256 + 16 * k)])
            t_v[pl.ds(16 * k, 16)] = acc

    def _pair(p, carry):
        @pl.when(p >= 1)
        def _drain():
            pltpu.make_async_copy(
                buf_a, out_hbm.at[b, i0 + 2 * p - 2, :], sem_a).wait()
            pltpu.make_async_copy(
                buf_b, out_hbm.at[b, i0 + 2 * p - 1, :], sem_b).wait()
        _stage_a(2 * p, t_a)
        _stage_a(2 * p + 1, t_b)
        for kc in range(_JCH):
            vb4 = vb4_v[pl.ds(16 * kc, 16)]
            nvs = [nvt_v[pl.ds(256 * r + 16 * kc, 16)]
                   for r in range(_Q + 1)]
            jbase = iota3 + (48 * kc)
            for t_v, buf in ((t_a, buf_a), (t_b, buf_b)):
                accs = []
                for c in range(4):
                    a = nvs[0] * plsc.load_gather(t_v, [vb4 + c])
                    for r in range(1, _Q + 1):
                        a = a + nvs[r] * plsc.load_gather(
                            t_v, [vb4 + (4 * r + c)])
                    accs.append(a)
                w = accs[3]
                for c in range(_DIM):
                    plsc.store_scatter(buf, [jbase + c], accs[c] / w)
        pltpu.async_copy(buf_a, out_hbm.at[b, i0 + 2 * p, :], sem_a)
        pltpu.async_copy(buf_b, out_hbm.at[b, i0 + 2 * p + 1, :], sem_b)
        return carry

    lax.fori_loop(0, _ROWS_PER_W // 2, _pair, 0)
    pltpu.make_async_copy(
        buf_a, out_hbm.at[b, i0 + _ROWS_PER_W - 2, :], sem_a).wait()
    pltpu.make_async_copy(
        buf_b, out_hbm.at[b, i0 + _ROWS_PER_W - 1, :], sem_b).wait()


def kernel(ctrl_pts, Nu_uv, Nv_uv, uspan_uv, vspan_uv):
    batch, m, n, dimp1 = ctrl_pts.shape
    ctrl2 = ctrl_pts.reshape(batch, m * n * dimp1)

    ub = (uspan_uv - _P).astype(jnp.int32)
    vb = (vspan_uv - _Q).astype(jnp.int32)
    ub256 = jnp.broadcast_to((ub * _NCOL)[:, None], (_OUT_U, 16)).reshape(-1)
    nurep = jnp.broadcast_to(Nu_uv.astype(jnp.float32)[:, :, None],
                             (_OUT_U, _P + 1, 16)).reshape(-1)
    vb4c = vb * 4
    nvtc = Nv_uv.astype(jnp.float32).T.reshape(-1)

    run = functools.partial(
        pl.kernel,
        mesh=plsc.VectorSubcoreMesh(core_axis_name="c", subcore_axis_name="s"),
        compiler_params=pltpu.CompilerParams(needs_layout_passes=False),
        out_type=jax.ShapeDtypeStruct((batch, _OUT_U, _DIM * _OUT_V),
                                      jnp.float32),
        scratch_types=[
            pltpu.VMEM((_M * _NCOL,), jnp.float32),
            pltpu.VMEM((_ROWS_PER_W * 16,), jnp.int32),
            pltpu.VMEM((_ROWS_PER_W * (_P + 1) * 16,), jnp.float32),
            pltpu.VMEM((_OUT_V,), jnp.int32),
            pltpu.VMEM(((_Q + 1) * _OUT_V,), jnp.float32),
            pltpu.VMEM((_NCOL,), jnp.float32),
            pltpu.VMEM((_NCOL,), jnp.float32),
            pltpu.VMEM((_DIM * _OUT_V,), jnp.float32),
            pltpu.VMEM((_DIM * _OUT_V,), jnp.float32),
            pltpu.SemaphoreType.DMA,
            pltpu.SemaphoreType.DMA,
        ],
    )(_sc_body)
    out3 = run(ctrl2, ub256, nurep, vb4c, nvtc)
    return out3.reshape(batch, _OUT_U, _OUT_V, dimp1 - 1)
